# baseline (speedup 1.0000x reference)
import functools

import jax
import jax.numpy as jnp
from jax import lax
from jax.experimental import pallas as pl
from jax.experimental.pallas import tpu as pltpu
from jax.experimental.pallas import tpu_sc as plsc

B = 128
V = 100000
CS = 1024
NCH = V // CS
TS = V - NCH * CS

W0 = 50048
W1 = 49920
WT = 32
CHK = 2176
NCHK = W0 // CHK

NEG_INF = float("-inf")


def _cumsum_lanes(p, width):
    k = 1
    while k < width:
        shifted = jnp.concatenate(
            [jnp.zeros((p.shape[0], k), p.dtype), p[:, : width - k]], axis=1
        )
        p = p + shifted
        k *= 2
    return p


def _process_chunk(x, base, width, carry, rand, idx, lp):
    p = jnp.exp(x)
    total = carry + _cumsum_lanes(p, width)
    prev_total = jnp.concatenate([carry, total[:, : width - 1]], axis=1)
    onehot = jnp.logical_and(total >= rand, prev_total < rand)
    col = base + jax.lax.broadcasted_iota(jnp.int32, (B, width), 1)
    has = jnp.any(onehot, axis=1, keepdims=True)
    idx_new = jnp.sum(jnp.where(onehot, col, 0), axis=1, keepdims=True)
    idx = jnp.where(has, idx_new, idx)
    lp = lp + jnp.sum(jnp.where(onehot, x, 0.0), axis=1, keepdims=True)
    return total[:, width - 1 :], idx, lp


def _search_kernel(x_hbm, rand_ref, idx_ref, lp_ref, chunk_ref, tail_ref, sem):
    rand = rand_ref[...]

    def cond(state):
        c, carry, _, _ = state
        return jnp.logical_and(c < NCH, jnp.logical_not(jnp.all(carry >= rand)))

    def body(state):
        c, carry, idx, lp = state
        copy = pltpu.make_async_copy(x_hbm.at[:, pl.ds(c * CS, CS)], chunk_ref, sem)
        copy.start()
        copy.wait()
        carry, idx, lp = _process_chunk(
            chunk_ref[...], c * CS, CS, carry, rand, idx, lp
        )
        return c + 1, carry, idx, lp

    init = (
        jnp.int32(0),
        jnp.zeros((B, 1), jnp.float32),
        jnp.full((B, 1), -1, jnp.int32),
        jnp.zeros((B, 1), jnp.float32),
    )
    _, carry, idx, lp = jax.lax.while_loop(cond, body, init)

    def tail(args):
        carry, idx, lp = args
        copy = pltpu.make_async_copy(x_hbm.at[:, pl.ds(NCH * CS, TS)], tail_ref, sem)
        copy.start()
        copy.wait()
        return _process_chunk(tail_ref[...], NCH * CS, TS, carry, rand, idx, lp)

    carry, idx, lp = jax.lax.cond(
        jnp.all(carry >= rand), lambda a: a, tail, (carry, idx, lp)
    )
    idx_ref[...] = idx
    lp_ref[...] = lp


_mesh = plsc.VectorSubcoreMesh(core_axis_name="c", subcore_axis_name="s")


@functools.partial(
    pl.kernel,
    mesh=_mesh,
    out_type=jax.ShapeDtypeStruct((B, V), jnp.float32),
    scratch_types=[
        pltpu.VMEM((8, CHK), jnp.float32),
        pltpu.VMEM((8, WT), jnp.float32),
        pltpu.VMEM_SHARED((8, W0), jnp.float32),
        pltpu.SemaphoreType.DMA,
    ],
)
def _sc_fill(out_hbm, fill_v, tail_v, shared_v, semA):
    c = lax.axis_index("c")
    s = lax.axis_index("s")
    w = c * 16 + s
    g = w // 2
    h = w - g * 2
    neg = jnp.full((16,), NEG_INF, jnp.float32)

    for r in range(8):
        def init_body(i, carry, r=r):
            fill_v[r, pl.ds(i * 16, 16)] = neg
            return carry

        lax.fori_loop(0, CHK // 16, init_body, 0)
        tail_v[r, pl.ds(0, 16)] = neg
        tail_v[r, pl.ds(16, 16)] = neg

    for rep in range(2):
        k = s + 16 * rep

        @pl.when(k < NCHK)
        def _stage(k=k):
            pltpu.sync_copy(fill_v, shared_v.at[:, pl.ds(k * CHK, CHK)])

    plsc.subcore_barrier()

    row0 = pl.multiple_of(g * 8, 8)

    @pl.when(h == 0)
    def _fill0():
        cp = pltpu.make_async_copy(
            shared_v, out_hbm.at[pl.ds(row0, 8), pl.ds(0, W0)], semA
        )
        cp.start()
        tp = pltpu.make_async_copy(
            tail_v, out_hbm.at[pl.ds(row0, 8), pl.ds(W0 + W1, WT)], semA
        )
        tp.start()
        cp.wait()
        tp.wait()

    @pl.when(h == 1)
    def _fill1():
        cp = pltpu.make_async_copy(
            shared_v.at[:, pl.ds(0, W1)],
            out_hbm.at[pl.ds(row0, 8), pl.ds(W0, W1)],
            semA,
        )
        cp.start()
        cp.wait()


def _patch_kernel(idx_sp, filled_ref, idxcol_ref, out_ref):
    g = pl.program_id(0)
    j = pl.program_id(1)
    t = jnp.maximum(idx_sp[g * 8 + j], 0) // 128
    idxb = idxcol_ref[...]
    col = t * 128 + jax.lax.broadcasted_iota(jnp.int32, (8, 128), 1)
    out_ref[...] = jnp.where(col == idxb, 0.0, NEG_INF)


def kernel(inputs, manualrand):
    idx, lp = pl.pallas_call(
        _search_kernel,
        in_specs=[
            pl.BlockSpec(memory_space=pl.ANY),
            pl.BlockSpec((B, 1), lambda: (0, 0)),
        ],
        out_specs=[
            pl.BlockSpec((B, 1), lambda: (0, 0)),
            pl.BlockSpec((B, 1), lambda: (0, 0)),
        ],
        out_shape=[
            jax.ShapeDtypeStruct((B, 1), jnp.int32),
            jax.ShapeDtypeStruct((B, 1), jnp.float32),
        ],
        scratch_shapes=[
            pltpu.VMEM((B, CS), jnp.float32),
            pltpu.VMEM((B, TS), jnp.float32),
            pltpu.SemaphoreType.DMA,
        ],
    )(inputs, manualrand)

    filled = _sc_fill()

    out = pl.pallas_call(
        _patch_kernel,
        grid_spec=pltpu.PrefetchScalarGridSpec(
            num_scalar_prefetch=1,
            grid=(16, 8),
            in_specs=[
                pl.BlockSpec(memory_space=pl.ANY),
                pl.BlockSpec(
                    (8, 1),
                    lambda g, j, sp: (g, 0),
                ),
            ],
            out_specs=pl.BlockSpec(
                (8, 128),
                lambda g, j, sp: (g, jnp.maximum(sp[g * 8 + j], 0) // 128),
            ),
        ),
        out_shape=jax.ShapeDtypeStruct((B, V), jnp.float32),
        input_output_aliases={1: 0},
        compiler_params=pltpu.CompilerParams(
            dimension_semantics=("arbitrary", "arbitrary")
        ),
    )(idx.reshape(B), filled, idx)
    return out, lp
